# Initial kernel scaffold; baseline (speedup 1.0000x reference)
import jax
import jax.numpy as jnp
from jax.experimental import pallas as pl


def kernel(x, edge_index, W, b):
    raise NotImplementedError("write your pallas kernel here")


# speedup vs baseline: 19.6295x; 19.6295x over previous
import functools

import jax
import jax.numpy as jnp
from jax import lax
from jax.experimental import pallas as pl
from jax.experimental.pallas import tpu as pltpu
from jax.experimental.pallas import tpu_sc as plsc

N = 10000
E = 320000
D = 128
NC = 2
NS = 16
NW = NC * NS
ET = E // NW
C = 80
NCH = ET // C
NP = 10240
STR = NP // NS

_MESH = plsc.VectorSubcoreMesh(core_axis_name="c", subcore_axis_name="s")


@functools.partial(
    pl.kernel,
    mesh=_MESH,
    out_type=(jax.ShapeDtypeStruct((N,), jnp.float32),
              jax.ShapeDtypeStruct((N,), jnp.float32)),
    scratch_types=[
        pltpu.VMEM((NCH, C), jnp.int32),
        pltpu.VMEM((C,), jnp.float32),
        pltpu.VMEM_SHARED((N,), jnp.float32),
    ],
)
def _deg_kernel(dst3_hbm, zerosf_hbm, pdeg0_hbm, pdeg1_hbm, didx, ones_v, acc):
    c = lax.axis_index("c")
    s = lax.axis_index("s")
    wid = s * NC + c

    pltpu.sync_copy(dst3_hbm.at[wid], didx)
    for j in range(C // 16):
        ones_v[pl.ds(j * 16, 16)] = jnp.ones((16,), jnp.float32)
    @pl.when(s == 0)
    def _():
        pltpu.sync_copy(zerosf_hbm, acc)
    plsc.subcore_barrier()

    def body(i, _):
        pltpu.sync_copy(ones_v, acc.at[didx.at[i]], add=True)
        return _
    lax.fori_loop(0, NCH, body, 0)

    plsc.subcore_barrier()
    @pl.when((s == 0) & (c == 0))
    def _():
        pltpu.sync_copy(acc, pdeg0_hbm)
    @pl.when((s == 0) & (c == 1))
    def _():
        pltpu.sync_copy(acc, pdeg1_hbm)


@functools.partial(
    pl.kernel,
    mesh=_MESH,
    out_type=jax.ShapeDtypeStruct((NC, NP, D), jnp.float32),
    scratch_types=[
        pltpu.VMEM((NCH, C), jnp.int32),
        pltpu.VMEM((NCH, C), jnp.int32),
        pltpu.VMEM((C, D), jnp.float32),
        pltpu.VMEM_SHARED((NP, D), jnp.float32),
        pltpu.SemaphoreType.DMA,
    ],
)
def _scatter_kernel(g_hbm, src3_hbm, dst3_hbm, zeros_hbm, ps_hbm,
                    sidx, didx, buf, acc, sem):
    c = lax.axis_index("c")
    s = lax.axis_index("s")
    wid = s * NC + c

    pltpu.sync_copy(src3_hbm.at[wid], sidx)
    pltpu.sync_copy(dst3_hbm.at[wid], didx)
    pltpu.sync_copy(zeros_hbm, acc.at[pl.ds(s * STR, STR)])
    plsc.subcore_barrier()

    def body(i, _):
        pltpu.async_copy(g_hbm.at[sidx.at[i]], buf, sem).wait()
        pltpu.sync_copy(buf, acc.at[didx.at[i]], add=True)
        return _
    lax.fori_loop(0, NCH, body, 0)

    plsc.subcore_barrier()
    pltpu.sync_copy(acc.at[pl.ds(s * STR, STR)],
                    ps_hbm.at[c, pl.ds(s * STR, STR)])


def _prep_body(p0_ref, p1_ref, x_ref, dinv_ref, g0_ref):
    deg = p0_ref[...] + p1_ref[...] + 1.0
    dinv = lax.rsqrt(deg)
    dinv_ref[...] = dinv
    g0_ref[...] = dinv * x_ref[...]


def _mid_body(p0_ref, p1_ref, g_ref, dinv_ref, out_ref):
    dinv = dinv_ref[...]
    out_ref[...] = dinv * dinv * (p0_ref[...] + p1_ref[...] + g_ref[...])


def _final_body(p0_ref, p1_ref, g_ref, dinv_ref, w_ref, b_ref, out_ref):
    h = dinv_ref[...] * (p0_ref[...] + p1_ref[...] + g_ref[...])
    out_ref[...] = (
        jnp.dot(h, w_ref[...], preferred_element_type=jnp.float32)
        + b_ref[...]
    )


_R = 2000


def _col_spec():
    return pl.BlockSpec((_R, 1), lambda i: (i, 0))


def _mat_spec():
    return pl.BlockSpec((_R, D), lambda i: (i, 0))


def _prep(p0c, p1c, x):
    return pl.pallas_call(
        _prep_body,
        grid=(N // _R,),
        in_specs=[_col_spec(), _col_spec(), _mat_spec()],
        out_specs=[_col_spec(), _mat_spec()],
        out_shape=[jax.ShapeDtypeStruct((N, 1), jnp.float32),
                   jax.ShapeDtypeStruct((N, D), jnp.float32)],
    )(p0c, p1c, x)


def _mid(p0, p1, g, dinv):
    return pl.pallas_call(
        _mid_body,
        grid=(N // _R,),
        in_specs=[_mat_spec(), _mat_spec(), _mat_spec(), _col_spec()],
        out_specs=_mat_spec(),
        out_shape=jax.ShapeDtypeStruct((N, D), jnp.float32),
    )(p0, p1, g, dinv)


def _final(p0, p1, g, dinv, W, b2):
    return pl.pallas_call(
        _final_body,
        grid=(N // _R,),
        in_specs=[_mat_spec(), _mat_spec(), _mat_spec(), _col_spec(),
                  pl.BlockSpec((D, D), lambda i: (0, 0)),
                  pl.BlockSpec((1, D), lambda i: (0, 0))],
        out_specs=_mat_spec(),
        out_shape=jax.ShapeDtypeStruct((N, D), jnp.float32),
    )(p0, p1, g, dinv, W, b2)


def kernel(x, edge_index, W, b):
    ei = edge_index.astype(jnp.int32)
    src3 = ei[0].reshape(NW, NCH, C)
    dst3 = ei[1].reshape(NW, NCH, C)
    zeros_f = jnp.zeros((N,), jnp.float32)
    zeros_m = jnp.zeros((STR, D), jnp.float32)

    pdeg0, pdeg1 = _deg_kernel(dst3, zeros_f)
    dinv, g0 = _prep(pdeg0.reshape(N, 1), pdeg1.reshape(N, 1), x)

    ps1 = _scatter_kernel(g0, src3, dst3, zeros_m)
    g1 = _mid(ps1[0, :N], ps1[1, :N], g0, dinv)

    ps2 = _scatter_kernel(g1, src3, dst3, zeros_m)
    out = _final(ps2[0, :N], ps2[1, :N], g1, dinv, W, b.reshape(1, D))
    return out


# speedup vs baseline: 22.3084x; 1.1365x over previous
import functools

import jax
import jax.numpy as jnp
from jax import lax
from jax.experimental import pallas as pl
from jax.experimental.pallas import tpu as pltpu
from jax.experimental.pallas import tpu_sc as plsc

N = 10000
E = 320000
D = 128
DH = D // 2
NC = 2
NS = 16
NW = NC * NS
ET = E // NW
C = 125
NCH = ET // C
K = 2
NP = 10240
STR = NP // NS

_MESH = plsc.VectorSubcoreMesh(core_axis_name="c", subcore_axis_name="s")


@functools.partial(
    pl.kernel,
    mesh=_MESH,
    out_type=(jax.ShapeDtypeStruct((N,), jnp.float32),
              jax.ShapeDtypeStruct((N,), jnp.float32)),
    scratch_types=[
        pltpu.VMEM((NCH, C), jnp.int32),
        pltpu.VMEM((C,), jnp.float32),
        pltpu.VMEM_SHARED((N,), jnp.float32),
    ],
)
def _deg_kernel(dst3_hbm, zerosf_hbm, pdeg0_hbm, pdeg1_hbm, didx, ones_v, acc):
    c = lax.axis_index("c")
    s = lax.axis_index("s")
    wid = s * NC + c

    pltpu.sync_copy(dst3_hbm.at[wid], didx)
    for j in range(C // 16):
        ones_v[pl.ds(j * 16, 16)] = jnp.ones((16,), jnp.float32)
    ones_v[pl.ds(C - 16, 16)] = jnp.ones((16,), jnp.float32)
    @pl.when(s == 0)
    def _():
        pltpu.sync_copy(zerosf_hbm, acc)
    plsc.subcore_barrier()

    def body(i, _):
        pltpu.sync_copy(ones_v, acc.at[didx.at[i]], add=True)
        return _
    lax.fori_loop(0, NCH, body, 0)

    plsc.subcore_barrier()
    @pl.when((s == 0) & (c == 0))
    def _():
        pltpu.sync_copy(acc, pdeg0_hbm)
    @pl.when((s == 0) & (c == 1))
    def _():
        pltpu.sync_copy(acc, pdeg1_hbm)


@functools.partial(
    pl.kernel,
    mesh=_MESH,
    out_type=jax.ShapeDtypeStruct((NC, NP, D), jnp.float32),
    scratch_types=[
        pltpu.VMEM((NCH, C), jnp.int32),
        pltpu.VMEM((NCH, C), jnp.int32),
        pltpu.VMEM((C, D), jnp.float32),
        pltpu.VMEM_SHARED((NP, D), jnp.float32),
        pltpu.SemaphoreType.DMA,
    ],
)
def _scatter_kernel(g_hbm, src3_hbm, dst3_hbm, zeros_hbm, ps_hbm,
                    sidx, didx, buf, acc, gsem):
    c = lax.axis_index("c")
    s = lax.axis_index("s")
    wid = s * NC + c

    pltpu.sync_copy(src3_hbm.at[wid], sidx)
    pltpu.sync_copy(dst3_hbm.at[wid], didx)

    pltpu.sync_copy(zeros_hbm, acc.at[pl.ds(s * STR, STR)])
    plsc.subcore_barrier()

    def body(i, _):
        pltpu.async_copy(g_hbm.at[sidx.at[i]], buf, gsem).wait()
        pltpu.sync_copy(buf, acc.at[didx.at[i]], add=True)
        return _
    lax.fori_loop(0, NCH, body, 0)

    plsc.subcore_barrier()
    pltpu.sync_copy(acc.at[pl.ds(s * STR, STR)],
                    ps_hbm.at[c, pl.ds(s * STR, STR)])


def _prep_body(p0_ref, p1_ref, x_ref, dinv_ref, rdinv_ref, g0_ref):
    deg = p0_ref[...] + p1_ref[...] + 1.0
    dinv = lax.rsqrt(deg)
    dinv_ref[...] = dinv
    rdinv_ref[...] = lax.sqrt(deg)
    g0_ref[...] = dinv * x_ref[...]


def _mid_body(p0_ref, p1_ref, g_ref, dinv_ref, out_ref):
    d2 = dinv_ref[...] * dinv_ref[...]
    out_ref[...] = d2 * (p0_ref[...] + p1_ref[...] + g_ref[...])


def _final_body(p0_ref, p1_ref, g_ref, dinv_ref, w_ref, b_ref, out_ref):
    h = dinv_ref[...] * (p0_ref[...] + p1_ref[...] + g_ref[...])
    out_ref[...] = (
        jnp.dot(h, w_ref[...], preferred_element_type=jnp.float32)
        + b_ref[...]
    )


_R = 2000


def _col_spec():
    return pl.BlockSpec((_R, 1), lambda i: (i, 0))


def _half_spec():
    return pl.BlockSpec((_R, DH), lambda i: (i, 0))


def _mat_spec():
    return pl.BlockSpec((_R, D), lambda i: (i, 0))


def _prep(p0c, p1c, x):
    return pl.pallas_call(
        _prep_body,
        grid=(N // _R,),
        in_specs=[_col_spec(), _col_spec(), _mat_spec()],
        out_specs=[_col_spec(), _col_spec(), _mat_spec()],
        out_shape=[jax.ShapeDtypeStruct((N, 1), jnp.float32),
                   jax.ShapeDtypeStruct((N, 1), jnp.float32),
                   jax.ShapeDtypeStruct((N, D), jnp.float32)],
    )(p0c, p1c, x)


def _mid(p0, p1, g, dinv):
    return pl.pallas_call(
        _mid_body,
        grid=(N // _R,),
        in_specs=[_mat_spec(), _mat_spec(), _mat_spec(), _col_spec()],
        out_specs=_mat_spec(),
        out_shape=jax.ShapeDtypeStruct((N, D), jnp.float32),
    )(p0, p1, g, dinv)


def _final(p0, p1, g, dinv, W, b2):
    return pl.pallas_call(
        _final_body,
        grid=(N // _R,),
        in_specs=[_mat_spec(), _mat_spec(), _mat_spec(), _col_spec(),
                  pl.BlockSpec((D, D), lambda i: (0, 0)),
                  pl.BlockSpec((1, D), lambda i: (0, 0))],
        out_specs=_mat_spec(),
        out_shape=jax.ShapeDtypeStruct((N, D), jnp.float32),
    )(p0, p1, g, dinv, W, b2)


def kernel(x, edge_index, W, b):
    ei = edge_index.astype(jnp.int32)
    src3 = ei[0].reshape(NW, NCH, C)
    dst3 = ei[1].reshape(NW, NCH, C)
    zeros_f = jnp.zeros((N,), jnp.float32)
    zeros_m = jnp.zeros((STR, D), jnp.float32)

    pdeg0, pdeg1 = _deg_kernel(dst3, zeros_f)
    dinv, rdinv, g0 = _prep(pdeg0.reshape(N, 1), pdeg1.reshape(N, 1), x)

    ps1 = _scatter_kernel(g0, src3, dst3, zeros_m)
    g1 = _mid(ps1[0, :N], ps1[1, :N], g0, dinv)

    ps2 = _scatter_kernel(g1, src3, dst3, zeros_m)
    out = _final(ps2[0, :N], ps2[1, :N], g1, dinv, W, b.reshape(1, D))
    return out


# speedup vs baseline: 23.0604x; 1.0337x over previous
import functools

import jax
import jax.numpy as jnp
from jax import lax
from jax.experimental import pallas as pl
from jax.experimental.pallas import tpu as pltpu
from jax.experimental.pallas import tpu_sc as plsc

N = 10000
E = 320000
D = 128
DH = D // 2
NC = 2
NS = 16
NW = NC * NS
ET = E // NW
C = 125
NCH = ET // C
K = 2
NP = 10240
STR = NP // NS

_MESH = plsc.VectorSubcoreMesh(core_axis_name="c", subcore_axis_name="s")


@functools.partial(
    pl.kernel,
    mesh=_MESH,
    out_type=(jax.ShapeDtypeStruct((N,), jnp.float32),
              jax.ShapeDtypeStruct((N,), jnp.float32)),
    scratch_types=[
        pltpu.VMEM((NCH, C), jnp.int32),
        pltpu.VMEM((C,), jnp.float32),
        pltpu.VMEM_SHARED((N,), jnp.float32),
    ],
)
def _deg_kernel(dst3_hbm, zerosf_hbm, pdeg0_hbm, pdeg1_hbm, didx, ones_v, acc):
    c = lax.axis_index("c")
    s = lax.axis_index("s")
    wid = s * NC + c

    pltpu.sync_copy(dst3_hbm.at[wid], didx)
    for j in range(C // 16):
        ones_v[pl.ds(j * 16, 16)] = jnp.ones((16,), jnp.float32)
    ones_v[pl.ds(C - 16, 16)] = jnp.ones((16,), jnp.float32)
    @pl.when(s == 0)
    def _():
        pltpu.sync_copy(zerosf_hbm, acc)
    plsc.subcore_barrier()

    def body(i, _):
        pltpu.sync_copy(ones_v, acc.at[didx.at[i]], add=True)
        return _
    lax.fori_loop(0, NCH, body, 0)

    plsc.subcore_barrier()
    @pl.when((s == 0) & (c == 0))
    def _():
        pltpu.sync_copy(acc, pdeg0_hbm)
    @pl.when((s == 0) & (c == 1))
    def _():
        pltpu.sync_copy(acc, pdeg1_hbm)


@functools.partial(
    pl.kernel,
    mesh=_MESH,
    out_type=jax.ShapeDtypeStruct((NC, NP, D), jnp.float32),
    scratch_types=[
        pltpu.VMEM((NCH, C), jnp.int32),
        pltpu.VMEM((NCH, C), jnp.int32),
        pltpu.VMEM((C, D), jnp.float32),
        pltpu.VMEM_SHARED((NP, D), jnp.float32),
        pltpu.SemaphoreType.DMA,
    ],
)
def _scatter_kernel(g_hbm, src3_hbm, dst3_hbm, zeros_hbm, ps_hbm,
                    sidx, didx, buf, acc, gsem):
    c = lax.axis_index("c")
    s = lax.axis_index("s")
    wid = s * NC + c

    pltpu.sync_copy(src3_hbm.at[wid], sidx)
    pltpu.sync_copy(dst3_hbm.at[wid], didx)

    pltpu.sync_copy(zeros_hbm, acc.at[pl.ds(s * STR, STR)])
    plsc.subcore_barrier()

    def body(i, _):
        pltpu.async_copy(g_hbm.at[sidx.at[i]], buf, gsem).wait()
        pltpu.sync_copy(buf, acc.at[didx.at[i]], add=True)
        return _
    lax.fori_loop(0, NCH, body, 0)

    plsc.subcore_barrier()
    pltpu.sync_copy(acc.at[pl.ds(s * STR, STR)],
                    ps_hbm.at[c, pl.ds(s * STR, STR)])


def _prep_body(p0_ref, p1_ref, x_ref, dinv_ref, rdinv_ref, g0_ref):
    deg = p0_ref[...] + p1_ref[...] + 1.0
    dinv = lax.rsqrt(deg)
    dinv_ref[...] = dinv
    rdinv_ref[...] = lax.sqrt(deg)
    g0_ref[...] = dinv * x_ref[...]


def _mid_body(p0_ref, p1_ref, g_ref, dinv_ref, out_ref):
    d2 = dinv_ref[...] * dinv_ref[...]
    out_ref[...] = d2 * (p0_ref[0] + p1_ref[0] + g_ref[...])


def _final_body(p0_ref, p1_ref, g_ref, dinv_ref, w_ref, b_ref, out_ref):
    h = dinv_ref[...] * (p0_ref[0] + p1_ref[0] + g_ref[...])
    out_ref[...] = (
        jnp.dot(h, w_ref[...], preferred_element_type=jnp.float32)
        + b_ref[...]
    )


_R = 2000


def _col_spec():
    return pl.BlockSpec((_R, 1), lambda i: (i, 0))


def _half_spec():
    return pl.BlockSpec((_R, DH), lambda i: (i, 0))


def _mat_spec():
    return pl.BlockSpec((_R, D), lambda i: (i, 0))


def _prep(p0c, p1c, x):
    return pl.pallas_call(
        _prep_body,
        grid=(N // _R,),
        in_specs=[_col_spec(), _col_spec(), _mat_spec()],
        out_specs=[_col_spec(), _col_spec(), _mat_spec()],
        out_shape=[jax.ShapeDtypeStruct((N, 1), jnp.float32),
                   jax.ShapeDtypeStruct((N, 1), jnp.float32),
                   jax.ShapeDtypeStruct((N, D), jnp.float32)],
    )(p0c, p1c, x)


def _ps_spec(core):
    return pl.BlockSpec((1, _R, D), lambda i, core=core: (core, i, 0))


def _mid(ps, g, dinv):
    return pl.pallas_call(
        _mid_body,
        grid=(N // _R,),
        in_specs=[_ps_spec(0), _ps_spec(1), _mat_spec(), _col_spec()],
        out_specs=_mat_spec(),
        out_shape=jax.ShapeDtypeStruct((N, D), jnp.float32),
    )(ps, ps, g, dinv)


def _final(ps, g, dinv, W, b2):
    return pl.pallas_call(
        _final_body,
        grid=(N // _R,),
        in_specs=[_ps_spec(0), _ps_spec(1), _mat_spec(), _col_spec(),
                  pl.BlockSpec((D, D), lambda i: (0, 0)),
                  pl.BlockSpec((1, D), lambda i: (0, 0))],
        out_specs=_mat_spec(),
        out_shape=jax.ShapeDtypeStruct((N, D), jnp.float32),
    )(ps, ps, g, dinv, W, b2)


def kernel(x, edge_index, W, b):
    ei = edge_index.astype(jnp.int32)
    src3 = ei[0].reshape(NW, NCH, C)
    dst3 = ei[1].reshape(NW, NCH, C)
    zeros_f = jnp.zeros((N,), jnp.float32)
    zeros_m = jnp.zeros((STR, D), jnp.float32)

    pdeg0, pdeg1 = _deg_kernel(dst3, zeros_f)
    dinv, rdinv, g0 = _prep(pdeg0.reshape(N, 1), pdeg1.reshape(N, 1), x)

    ps1 = _scatter_kernel(g0, src3, dst3, zeros_m)
    g1 = _mid(ps1, g0, dinv)

    ps2 = _scatter_kernel(g1, src3, dst3, zeros_m)
    out = _final(ps2, g1, dinv, W, b.reshape(1, D))
    return out


# speedup vs baseline: 23.1040x; 1.0019x over previous
import functools

import jax
import jax.numpy as jnp
from jax import lax
from jax.experimental import pallas as pl
from jax.experimental.pallas import tpu as pltpu
from jax.experimental.pallas import tpu_sc as plsc

N = 10000
E = 320000
D = 128
NC = 2
NS = 16
NW = NC * NS
ET = E // NW
C = 125
NCH = ET // C
NP = 10240
STR = NP // NS

_MESH = plsc.VectorSubcoreMesh(core_axis_name="c", subcore_axis_name="s")


@functools.partial(
    pl.kernel,
    mesh=_MESH,
    out_type=(jax.ShapeDtypeStruct((N,), jnp.float32),
              jax.ShapeDtypeStruct((N,), jnp.float32)),
    scratch_types=[
        pltpu.VMEM((NCH, C), jnp.int32),
        pltpu.VMEM((C,), jnp.float32),
        pltpu.VMEM_SHARED((N,), jnp.float32),
    ],
)
def _deg_kernel(dst3_hbm, zerosf_hbm, pdeg0_hbm, pdeg1_hbm, didx, ones_v, acc):
    c = lax.axis_index("c")
    s = lax.axis_index("s")
    wid = s * NC + c

    pltpu.sync_copy(dst3_hbm.at[wid], didx)
    for j in range(C // 16):
        ones_v[pl.ds(j * 16, 16)] = jnp.ones((16,), jnp.float32)
    ones_v[pl.ds(C - 16, 16)] = jnp.ones((16,), jnp.float32)
    @pl.when(s == 0)
    def _():
        pltpu.sync_copy(zerosf_hbm, acc)
    plsc.subcore_barrier()

    def body(i, _):
        pltpu.sync_copy(ones_v, acc.at[didx.at[i]], add=True)
        return _
    lax.fori_loop(0, NCH, body, 0)

    plsc.subcore_barrier()
    @pl.when((s == 0) & (c == 0))
    def _():
        pltpu.sync_copy(acc, pdeg0_hbm)
    @pl.when((s == 0) & (c == 1))
    def _():
        pltpu.sync_copy(acc, pdeg1_hbm)


@functools.partial(
    pl.kernel,
    mesh=_MESH,
    out_type=jax.ShapeDtypeStruct((NC, NP, D), jnp.float32),
    scratch_types=[
        pltpu.VMEM((NCH, C), jnp.int32),
        pltpu.VMEM((NCH, C), jnp.int32),
        pltpu.VMEM((C, D), jnp.float32),
        pltpu.VMEM_SHARED((NP, D), jnp.float32),
        pltpu.SemaphoreType.DMA,
    ],
)
def _scatter_kernel(g_hbm, src3_hbm, dst3_hbm, zeros_hbm, ps_hbm,
                    sidx, didx, buf, acc, gsem):
    c = lax.axis_index("c")
    s = lax.axis_index("s")
    wid = s * NC + c

    pltpu.sync_copy(src3_hbm.at[wid], sidx)
    pltpu.sync_copy(dst3_hbm.at[wid], didx)

    pltpu.sync_copy(zeros_hbm, acc.at[pl.ds(s * STR, STR)])
    plsc.subcore_barrier()

    def body(i, _):
        pltpu.async_copy(g_hbm.at[sidx.at[i]], buf, gsem).wait()
        pltpu.sync_copy(buf, acc.at[didx.at[i]], add=True)
        return _
    lax.fori_loop(0, NCH, body, 0)

    plsc.subcore_barrier()
    pltpu.sync_copy(acc.at[pl.ds(s * STR, STR)],
                    ps_hbm.at[c, pl.ds(s * STR, STR)])


def _prep_body(p0_ref, p1_ref, x_ref, dinv_ref, g0_ref):
    deg = p0_ref[...] + p1_ref[...] + 1.0
    dinv = lax.rsqrt(deg)
    dinv_ref[...] = dinv
    g0_ref[...] = dinv * x_ref[...]


def _mid_body(p0_ref, p1_ref, g_ref, dinv_ref, out_ref):
    d2 = dinv_ref[...] * dinv_ref[...]
    out_ref[...] = d2 * (p0_ref[0] + p1_ref[0] + g_ref[...])


def _final_body(p0_ref, p1_ref, g_ref, dinv_ref, w_ref, b_ref, out_ref):
    h = dinv_ref[...] * (p0_ref[0] + p1_ref[0] + g_ref[...])
    out_ref[...] = (
        jnp.dot(h, w_ref[...], preferred_element_type=jnp.float32)
        + b_ref[...]
    )


_R = 2000


def _col_spec():
    return pl.BlockSpec((_R, 1), lambda i: (i, 0))


def _mat_spec():
    return pl.BlockSpec((_R, D), lambda i: (i, 0))


def _prep(p0c, p1c, x):
    return pl.pallas_call(
        _prep_body,
        grid=(N // _R,),
        in_specs=[_col_spec(), _col_spec(), _mat_spec()],
        out_specs=[_col_spec(), _mat_spec()],
        out_shape=[jax.ShapeDtypeStruct((N, 1), jnp.float32),
                   jax.ShapeDtypeStruct((N, D), jnp.float32)],
    )(p0c, p1c, x)


def _ps_spec(core):
    return pl.BlockSpec((1, _R, D), lambda i, core=core: (core, i, 0))


def _mid(ps, g, dinv):
    return pl.pallas_call(
        _mid_body,
        grid=(N // _R,),
        in_specs=[_ps_spec(0), _ps_spec(1), _mat_spec(), _col_spec()],
        out_specs=_mat_spec(),
        out_shape=jax.ShapeDtypeStruct((N, D), jnp.float32),
    )(ps, ps, g, dinv)


def _final(ps, g, dinv, W, b2):
    return pl.pallas_call(
        _final_body,
        grid=(N // _R,),
        in_specs=[_ps_spec(0), _ps_spec(1), _mat_spec(), _col_spec(),
                  pl.BlockSpec((D, D), lambda i: (0, 0)),
                  pl.BlockSpec((1, D), lambda i: (0, 0))],
        out_specs=_mat_spec(),
        out_shape=jax.ShapeDtypeStruct((N, D), jnp.float32),
    )(ps, ps, g, dinv, W, b2)


def kernel(x, edge_index, W, b):
    ei = edge_index.astype(jnp.int32)
    src3 = ei[0].reshape(NW, NCH, C)
    dst3 = ei[1].reshape(NW, NCH, C)
    zeros_f = jnp.zeros((N,), jnp.float32)
    zeros_m = jnp.zeros((STR, D), jnp.float32)

    pdeg0, pdeg1 = _deg_kernel(dst3, zeros_f)
    dinv, g0 = _prep(pdeg0.reshape(N, 1), pdeg1.reshape(N, 1), x)

    ps1 = _scatter_kernel(g0, src3, dst3, zeros_m)
    g1 = _mid(ps1, g0, dinv)

    ps2 = _scatter_kernel(g1, src3, dst3, zeros_m)
    out = _final(ps2, g1, dinv, W, b.reshape(1, D))
    return out
